# baseline (speedup 1.0000x reference)
import functools

import jax
import jax.numpy as jnp
from jax.experimental import pallas as pl
from jax.experimental.pallas import tpu as pltpu

S, D, E, TOPK, FF = 2048, 768, 8, 2, 2048
TB = 256
NB = (S * TOPK) // TB + E
NPP = NB * TB


def _gating_body(x_ref, wr_ref, i1_ref, i2_ref, w1_ref, w2_ref,
                 counts_ref, loss_ref):
    x = x_ref[...]
    wr = wr_ref[...]
    logits = jax.lax.dot_general(
        x, wr, (((1,), (1,)), ((), ())),
        preferred_element_type=jnp.float32,
    )
    m = jnp.max(logits, axis=-1, keepdims=True)
    p = jnp.exp(logits - m)
    p = p / jnp.sum(p, axis=-1, keepdims=True)

    g1 = jnp.max(p, axis=-1, keepdims=True)
    i1 = jnp.argmax(p, axis=-1, keepdims=True)
    lanes = jax.lax.broadcasted_iota(jnp.int32, (S, E), 1)
    p2 = jnp.where(lanes == i1, -jnp.inf, p)
    g2 = jnp.max(p2, axis=-1, keepdims=True)
    i2 = jnp.argmax(p2, axis=-1, keepdims=True)
    denom = g1 + g2 + 1e-8
    w1_ref[...] = g1 / denom
    w2_ref[...] = g2 / denom
    i1_ref[...] = i1.astype(jnp.int32)
    i2_ref[...] = i2.astype(jnp.int32)

    sel1 = (lanes == i1).astype(jnp.float32)
    sel2 = (lanes == i2).astype(jnp.float32)
    counts = jnp.sum(sel1 + sel2, axis=0, keepdims=True)
    counts_ref[...] = counts
    usage = counts / jnp.sum(counts)
    mean = jnp.mean(usage)
    var = jnp.sum((usage - mean) ** 2) / (E - 1)
    cv2 = (var / (mean + 1e-8)) ** 2
    loss_ref[0, 0] = cv2


def _ffn_body(ea_ref, xs_ref, wg_ref, wu_ref, wd_ref, wgt_ref, y_ref):
    x = xs_ref[...]
    g = jax.lax.dot_general(
        x, wg_ref[0], (((1,), (1,)), ((), ())),
        preferred_element_type=jnp.float32)
    u = jax.lax.dot_general(
        x, wu_ref[0], (((1,), (1,)), ((), ())),
        preferred_element_type=jnp.float32)
    act = g * (u * jax.nn.sigmoid(u))
    y = jax.lax.dot_general(
        act, wd_ref[0], (((1,), (1,)), ((), ())),
        preferred_element_type=jnp.float32)
    y_ref[...] = wgt_ref[...] * y


@jax.jit
def kernel(x, Wg, Wu, Wd, Wr):
    b, s, d = x.shape
    x2 = x.reshape(s, d)

    i1, i2, w1, w2, counts, loss = pl.pallas_call(
        _gating_body,
        out_shape=(
            jax.ShapeDtypeStruct((S, 1), jnp.int32),
            jax.ShapeDtypeStruct((S, 1), jnp.int32),
            jax.ShapeDtypeStruct((S, 1), jnp.float32),
            jax.ShapeDtypeStruct((S, 1), jnp.float32),
            jax.ShapeDtypeStruct((1, E), jnp.float32),
            jax.ShapeDtypeStruct((1, 1), jnp.float32),
        ),
        in_specs=[
            pl.BlockSpec((S, D), lambda: (0, 0)),
            pl.BlockSpec((E, D), lambda: (0, 0)),
        ],
        out_specs=(
            pl.BlockSpec((S, 1), lambda: (0, 0)),
            pl.BlockSpec((S, 1), lambda: (0, 0)),
            pl.BlockSpec((S, 1), lambda: (0, 0)),
            pl.BlockSpec((S, 1), lambda: (0, 0)),
            pl.BlockSpec((1, E), lambda: (0, 0)),
            pl.BlockSpec(memory_space=pltpu.SMEM),
        ),
    )(x2, Wr)

    cnt = counts[0].astype(jnp.int32)
    eid = jnp.concatenate([i1[:, 0], i2[:, 0]])
    tok = jnp.concatenate([jnp.arange(S, dtype=jnp.int32)] * 2)
    wgt = jnp.concatenate([w1[:, 0], w2[:, 0]])
    order = jnp.argsort(eid, stable=True)
    eid_s = eid[order]
    coff = jnp.cumsum(cnt) - cnt
    pcnt = ((cnt + TB - 1) // TB) * TB
    poff = jnp.cumsum(pcnt) - pcnt
    rank = (jnp.arange(2 * S, dtype=jnp.int32) - coff[eid_s] + poff[eid_s])
    tok_p = jnp.zeros((NPP,), jnp.int32).at[rank].set(tok[order])
    wgt_p = jnp.zeros((NPP,), jnp.float32).at[rank].set(wgt[order])
    pos = jnp.zeros((2 * S,), jnp.int32).at[order].set(rank)
    bend = jnp.cumsum(pcnt) // TB
    block_expert = jnp.minimum(
        jnp.searchsorted(bend, jnp.arange(NB, dtype=jnp.int32), side="right"),
        E - 1).astype(jnp.int32)

    xs = x2[tok_p]

    y = pl.pallas_call(
        _ffn_body,
        grid_spec=pltpu.PrefetchScalarGridSpec(
            num_scalar_prefetch=1,
            grid=(NB,),
            in_specs=[
                pl.BlockSpec((TB, D), lambda i, ea: (i, 0)),
                pl.BlockSpec((1, FF, D), lambda i, ea: (ea[i], 0, 0)),
                pl.BlockSpec((1, FF, D), lambda i, ea: (ea[i], 0, 0)),
                pl.BlockSpec((1, D, FF), lambda i, ea: (ea[i], 0, 0)),
                pl.BlockSpec((TB, 1), lambda i, ea: (i, 0)),
            ],
            out_specs=pl.BlockSpec((TB, D), lambda i, ea: (i, 0)),
        ),
        out_shape=jax.ShapeDtypeStruct((NPP, D), jnp.float32),
    )(block_expert, xs, Wg, Wu, Wd, wgt_p.reshape(NPP, 1))

    out = y[pos[:S]] + y[pos[S:]]
    return out.reshape(b, s, d), loss.reshape(())
